# Initial kernel scaffold; baseline (speedup 1.0000x reference)
import jax
import jax.numpy as jnp
from jax.experimental import pallas as pl


def kernel(x, params):
    raise NotImplementedError("write your pallas kernel here")


# speedup vs baseline: 6.4485x; 6.4485x over previous
import functools

import numpy as np
import jax
import jax.numpy as jnp
from jax import lax
from jax.experimental import pallas as pl
from jax.experimental.pallas import tpu as pltpu
from jax.experimental.pallas import tpu_sc as plsc

_BN_C = np.sqrt(np.float32(1.0 + 1e-5)).astype(np.float32)


def _fps(xr, n_out):
    B, _, S, L = xr.shape
    N = S * L

    def body(xr_ref, idx_ref, nx_ref):
        X = xr_ref[:, 0]
        Y = xr_ref[:, 1]
        Z = xr_ref[:, 2]
        iota3 = (lax.broadcasted_iota(jnp.int32, (B, S, L), 1) * L
                 + lax.broadcasted_iota(jnp.int32, (B, S, L), 2))
        row_n = lax.broadcasted_iota(jnp.int32, (B, n_out), 0)
        lane1 = lax.broadcasted_iota(jnp.int32, (B, n_out), 1)

        def step(t, carry):
            dist, f, C, NX, NY, NZ = carry
            C = jnp.where(lane1 == t, f, C)
            m3 = iota3 == f[:, :, None]
            cx = jnp.sum(jnp.sum(jnp.where(m3, X, 0.0), axis=2), axis=1)[:, None]
            cy = jnp.sum(jnp.sum(jnp.where(m3, Y, 0.0), axis=2), axis=1)[:, None]
            cz = jnp.sum(jnp.sum(jnp.where(m3, Z, 0.0), axis=2), axis=1)[:, None]
            NX = jnp.where(lane1 == t, cx, NX)
            NY = jnp.where(lane1 == t, cy, NY)
            NZ = jnp.where(lane1 == t, cz, NZ)
            dx = X - cx[:, :, None]
            dy = Y - cy[:, :, None]
            dz = Z - cz[:, :, None]
            d = dx * dx + dy * dy + dz * dz
            dist = jnp.minimum(dist, d)
            mx = jnp.max(jnp.max(dist, axis=2), axis=1)[:, None]
            f_new = jnp.min(jnp.min(
                jnp.where(dist == mx[:, :, None], iota3, N), axis=2), axis=1)[:, None]
            return (dist, f_new, C, NX, NY, NZ)

        zi = jnp.where(row_n < 0, row_n, 0)
        zf = jnp.where(row_n < 0, 1.0, 0.0)
        init = (1e10 + 0.0 * X, jnp.zeros((B, 1), jnp.int32), zi, zf, zf, zf)
        _, _, C, NX, NY, NZ = lax.fori_loop(0, n_out, step, init)
        idx_ref[...] = C
        nx_ref[:, 0, :] = NX
        nx_ref[:, 1, :] = NY
        nx_ref[:, 2, :] = NZ

    return pl.pallas_call(
        body,
        out_shape=[jax.ShapeDtypeStruct((B, n_out), jnp.int32),
                   jax.ShapeDtypeStruct((B, 3, n_out), jnp.float32)],
    )(xr)


def _ball_query(new_xyz, xyz, xyzT, radius, nsample):
    B, S, _ = new_xyz.shape
    N = xyz.shape[1]
    r2 = radius ** 2

    def body(a_ref, b_ref, bt_ref, o_ref):
        a = a_ref[0]
        bm = b_ref[0]
        bt = bt_ref[0]
        sa = (a[:, 0:1] * a[:, 0:1] + a[:, 1:2] * a[:, 1:2]
              + a[:, 2:3] * a[:, 2:3])
        sb = (bt[0:1] * bt[0:1] + bt[1:2] * bt[1:2]
              + bt[2:3] * bt[2:3])
        d = lax.dot_general(a, bm, (((1,), (1,)), ((), ())),
                            preferred_element_type=jnp.float32)
        sq = sa + sb - 2.0 * d
        col = lax.broadcasted_iota(jnp.int32, (S, N), 1)
        keys = jnp.where(sq > r2, N, col)
        rowO = lax.broadcasted_iota(jnp.int32, (S, nsample), 0)
        colO = lax.broadcasted_iota(jnp.int32, (S, nsample), 1)
        O0 = jnp.where(rowO < 0, rowO, N)

        def cond(c):
            _, _, j, have = c
            return jnp.logical_and(j < nsample, have)

        def step(c):
            keys, O, j, _ = c
            m = jnp.min(keys, axis=1, keepdims=True)
            O = jnp.where(colO == j, m, O)
            keys = jnp.where(keys == m, N, keys)
            return (keys, O, j + 1, jnp.min(keys) < N)

        _, O, _, _ = lax.while_loop(
            cond, step, (keys, O0, jnp.int32(0), jnp.min(keys) < N))
        first = O[:, 0:1]
        o_ref[0] = jnp.where(O == N, first, O)

    return pl.pallas_call(
        body,
        grid=(B,),
        in_specs=[pl.BlockSpec((1, S, 3), lambda b: (b, 0, 0)),
                  pl.BlockSpec((1, N, 3), lambda b: (b, 0, 0)),
                  pl.BlockSpec((1, 3, N), lambda b: (b, 0, 0))],
        out_specs=pl.BlockSpec((1, S, nsample), lambda b: (b, 0, 0)),
        out_shape=jax.ShapeDtypeStruct((B, S, nsample), jnp.int32),
    )(new_xyz, xyz, xyzT)


def _sc_gather(table, idx):
    V, D = table.shape
    R = idx.shape[0]
    NC, NS, CH = 2, 16, 128
    NW = NC * NS
    npc = R // (NW * CH)
    assert R == NW * CH * npc, (R, D)
    mesh = plsc.VectorSubcoreMesh(core_axis_name="c", subcore_axis_name="s")

    @functools.partial(
        pl.kernel, mesh=mesh,
        out_type=jax.ShapeDtypeStruct((R, D), jnp.float32),
        scratch_types=[pltpu.VMEM((CH,), jnp.int32),
                       pltpu.VMEM((CH, D), jnp.float32),
                       pltpu.SemaphoreType.DMA],
    )
    def k(tab, ix, out, ix_v, rows_v, sem):
        wid = lax.axis_index("s") * NC + lax.axis_index("c")

        def chunk(c, carry):
            base = pl.multiple_of((wid * npc + c) * CH, CH)
            pltpu.sync_copy(ix.at[pl.ds(base, CH)], ix_v)
            pltpu.async_copy(tab.at[ix_v], rows_v, sem).wait()
            pltpu.sync_copy(rows_v, out.at[pl.ds(base, CH)])
            return carry

        lax.fori_loop(0, npc, chunk, 0)

    return k(table, idx)


def _bn(h, gamma, beta):
    return (h / _BN_C) * gamma + beta


def _mlp_pool1(grouped, new_xyz, w):
    B, RS, _ = grouped.shape
    S = new_xyz.shape[1]
    K = RS // S
    (w1, b1, g1, e1), (w2, b2, g2, e2), (w3, b3, g3, e3) = w
    co = w3.shape[1]

    def body(g_ref, nx_ref, w1r, b1r, g1r, e1r, w2r, b2r, g2r, e2r,
             w3r, b3r, g3r, e3r, o_ref):
        nx = nx_ref[0]
        W1, W2, W3 = w1r[...], w2r[...], w3r[...]
        B1, G1, E1 = b1r[...], g1r[...], e1r[...]
        B2, G2, E2 = b2r[...], g2r[...], e2r[...]
        B3, G3, E3 = b3r[...], g3r[...], e3r[...]

        def one(k):
            g = g_ref[0, pl.ds(k * S, S), :]
            xc = g[:, 0:3] - nx
            h = jax.nn.relu(_bn(lax.dot_general(
                xc, W1, (((1,), (0,)), ((), ())),
                preferred_element_type=jnp.float32) + B1, G1, E1))
            h = jax.nn.relu(_bn(lax.dot_general(
                h, W2, (((1,), (0,)), ((), ())),
                preferred_element_type=jnp.float32) + B2, G2, E2))
            h = jax.nn.relu(_bn(lax.dot_general(
                h, W3, (((1,), (0,)), ((), ())),
                preferred_element_type=jnp.float32) + B3, G3, E3))
            return h

        o_ref[0] = lax.fori_loop(
            1, K, lambda k, acc: jnp.maximum(acc, one(k)), one(0))

    wspecs = []
    wargs = []
    for (W, bb, gg, ee) in ((w1, b1, g1, e1), (w2, b2, g2, e2), (w3, b3, g3, e3)):
        for arr in (W, bb, gg, ee):
            wspecs.append(pl.BlockSpec(arr.shape, lambda b: (0,) * arr.ndim))
            wargs.append(arr)
    return pl.pallas_call(
        body,
        grid=(B,),
        in_specs=[pl.BlockSpec((1, RS, grouped.shape[2]), lambda b: (b, 0, 0)),
                  pl.BlockSpec((1, S, 3), lambda b: (b, 0, 0))] + wspecs,
        out_specs=pl.BlockSpec((1, S, co), lambda b: (b, 0, 0)),
        out_shape=jax.ShapeDtypeStruct((B, S, co), jnp.float32),
    )(grouped, new_xyz, *wargs)


def _mlp_pool2(grouped, new_xyz, w):
    B, RS, DD = grouped.shape
    S = new_xyz.shape[1]
    K = RS // S
    (w1x, w1f, b1, g1, e1), (w2, b2, g2, e2), (w3, b3, g3, e3) = w
    co = w3.shape[1]

    def body(g_ref, nx_ref, w1xr, w1fr, b1r, g1r, e1r, w2r, b2r, g2r, e2r,
             w3r, b3r, g3r, e3r, o_ref):
        nx = nx_ref[0]
        W1x, W1f = w1xr[...], w1fr[...]
        W2, W3 = w2r[...], w3r[...]
        B1, G1, E1 = b1r[...], g1r[...], e1r[...]
        B2, G2, E2 = b2r[...], g2r[...], e2r[...]
        B3, G3, E3 = b3r[...], g3r[...], e3r[...]

        def one(k):
            g = g_ref[0, pl.ds(k * S, S), :]
            F = g[:, 0:128]
            xc = g[:, 128:131] - nx
            h = lax.dot_general(xc, W1x, (((1,), (0,)), ((), ())),
                                preferred_element_type=jnp.float32)
            h = h + lax.dot_general(F, W1f, (((1,), (0,)), ((), ())),
                                    preferred_element_type=jnp.float32)
            h = jax.nn.relu(_bn(h + B1, G1, E1))
            h = jax.nn.relu(_bn(lax.dot_general(
                h, W2, (((1,), (0,)), ((), ())),
                preferred_element_type=jnp.float32) + B2, G2, E2))
            h = jax.nn.relu(_bn(lax.dot_general(
                h, W3, (((1,), (0,)), ((), ())),
                preferred_element_type=jnp.float32) + B3, G3, E3))
            return h

        o_ref[0] = lax.fori_loop(
            1, K, lambda k, acc: jnp.maximum(acc, one(k)), one(0))

    wspecs = []
    wargs = []
    for arr in (w1x, w1f, b1, g1, e1, w2, b2, g2, e2, w3, b3, g3, e3):
        wspecs.append(pl.BlockSpec(arr.shape, lambda b: (0,) * arr.ndim))
        wargs.append(arr)
    return pl.pallas_call(
        body,
        grid=(B,),
        in_specs=[pl.BlockSpec((1, RS, DD), lambda b: (b, 0, 0)),
                  pl.BlockSpec((1, S, 3), lambda b: (b, 0, 0))] + wspecs,
        out_specs=pl.BlockSpec((1, S, co), lambda b: (b, 0, 0)),
        out_shape=jax.ShapeDtypeStruct((B, S, co), jnp.float32),
    )(grouped, new_xyz, *wargs)


def _sa3(l2_xyz, l2_points, w):
    B, S, _ = l2_xyz.shape
    (w1x, w1f, b1, g1, e1), (w2, b2, g2, e2), (w3, b3, g3, e3) = w
    co = w3.shape[1]

    def body(x_ref, p_ref, w1xr, w1fr, b1r, g1r, e1r, w2r, b2r, g2r, e2r,
             w3r, b3r, g3r, e3r, o_ref):
        xyz = x_ref[0]
        pts = p_ref[0]
        h = lax.dot_general(xyz, w1xr[...], (((1,), (0,)), ((), ())),
                            preferred_element_type=jnp.float32)
        h = h + lax.dot_general(pts, w1fr[...], (((1,), (0,)), ((), ())),
                                preferred_element_type=jnp.float32)
        h = jax.nn.relu(_bn(h + b1r[...], g1r[...], e1r[...]))
        h = jax.nn.relu(_bn(lax.dot_general(
            h, w2r[...], (((1,), (0,)), ((), ())),
            preferred_element_type=jnp.float32) + b2r[...], g2r[...], e2r[...]))
        h = jax.nn.relu(_bn(lax.dot_general(
            h, w3r[...], (((1,), (0,)), ((), ())),
            preferred_element_type=jnp.float32) + b3r[...], g3r[...], e3r[...]))
        o_ref[0] = jnp.max(h, axis=0, keepdims=True)

    wspecs = []
    wargs = []
    for arr in (w1x, w1f, b1, g1, e1, w2, b2, g2, e2, w3, b3, g3, e3):
        wspecs.append(pl.BlockSpec(arr.shape, lambda b: (0,) * arr.ndim))
        wargs.append(arr)
    return pl.pallas_call(
        body,
        grid=(B,),
        in_specs=[pl.BlockSpec((1, S, 3), lambda b: (b, 0, 0)),
                  pl.BlockSpec((1, S, l2_points.shape[2]), lambda b: (b, 0, 0))]
                 + wspecs,
        out_specs=pl.BlockSpec((1, 1, co), lambda b: (b, 0, 0)),
        out_shape=jax.ShapeDtypeStruct((B, 1, co), jnp.float32),
    )(l2_xyz, l2_points, *wargs)


def _prep_layer1(lp):
    out = []
    for (W, b, gm, bt) in lp:
        co = W.shape[0]
        out.append((W.T, b.reshape(1, co), gm.reshape(1, co), bt.reshape(1, co)))
    return out


def _prep_layer_split(lp, nx):
    (W1, b1, gm1, bt1) = lp[0]
    co1 = W1.shape[0]
    if nx == 0:
        w1x = W1[:, 0:3].T
        w1f = W1[:, 3:].T
    else:
        w1x = W1[:, 0:3].T
        w1f = W1[:, 3:].T
    first = (w1x, w1f, b1.reshape(1, co1), gm1.reshape(1, co1), bt1.reshape(1, co1))
    rest = []
    for (W, b, gm, bt) in lp[1:]:
        co = W.shape[0]
        rest.append((W.T, b.reshape(1, co), gm.reshape(1, co), bt.reshape(1, co)))
    return (first, rest[0], rest[1])


def kernel(x, params):
    B, N, _ = x.shape
    p1, p2, p3 = params

    xT = x.transpose(0, 2, 1)
    xr1 = xT.reshape(B, 3, 8, N // 8)

    fps1_idx, nxyzT1 = _fps(xr1, 512)
    del fps1_idx
    new_xyz1 = nxyzT1.transpose(0, 2, 1)
    idx1 = _ball_query(new_xyz1, x, xT, 0.03, 32)

    table1 = jnp.pad(x.reshape(B * N, 3), ((0, 0), (0, 125)))
    gi1 = (jnp.minimum(idx1, N - 1).transpose(0, 2, 1)
           + (jnp.arange(B, dtype=jnp.int32) * N)[:, None, None])
    g1 = _sc_gather(table1, gi1.reshape(-1).astype(jnp.int32))
    grouped1 = g1.reshape(B, 32 * 512, 128)
    l1_points = _mlp_pool1(grouped1, new_xyz1, _prep_layer1(p1))

    xr2 = nxyzT1.reshape(B, 3, 8, 64)
    fps2_idx, nxyzT2 = _fps(xr2, 128)
    del fps2_idx
    new_xyz2 = nxyzT2.transpose(0, 2, 1)
    idx2 = _ball_query(new_xyz2, new_xyz1, nxyzT1, 0.06, 64)

    table2 = jnp.concatenate(
        [l1_points.reshape(B * 512, 128),
         new_xyz1.reshape(B * 512, 3),
         jnp.zeros((B * 512, 125), jnp.float32)], axis=1)
    gi2 = (jnp.minimum(idx2, 511).transpose(0, 2, 1)
           + (jnp.arange(B, dtype=jnp.int32) * 512)[:, None, None])
    g2 = _sc_gather(table2, gi2.reshape(-1).astype(jnp.int32))
    grouped2 = g2.reshape(B, 64 * 128, 256)
    l2_points = _mlp_pool2(grouped2, new_xyz2, _prep_layer_split(p2, 0))

    out = _sa3(new_xyz2, l2_points, _prep_layer_split(p3, 0))
    return out.reshape(B, 1024)


# speedup vs baseline: 6.4509x; 1.0004x over previous
import functools

import numpy as np
import jax
import jax.numpy as jnp
from jax import lax
from jax.experimental import pallas as pl
from jax.experimental.pallas import tpu as pltpu
from jax.experimental.pallas import tpu_sc as plsc

_BN_C = np.sqrt(np.float32(1.0 + 1e-5)).astype(np.float32)


def _fps(xr, n_out):
    B, _, S, L = xr.shape
    N = S * L

    def body(xr_ref, idx_ref, nx_ref):
        X = xr_ref[:, 0]
        Y = xr_ref[:, 1]
        Z = xr_ref[:, 2]
        iota3 = (lax.broadcasted_iota(jnp.int32, (B, S, L), 1) * L
                 + lax.broadcasted_iota(jnp.int32, (B, S, L), 2))
        row_n = lax.broadcasted_iota(jnp.int32, (B, n_out), 0)
        lane1 = lax.broadcasted_iota(jnp.int32, (B, n_out), 1)

        def step(t, carry):
            dist, f, C, NX, NY, NZ = carry
            C = jnp.where(lane1 == t, f, C)
            m3 = iota3 == f[:, :, None]
            cx = jnp.sum(jnp.sum(jnp.where(m3, X, 0.0), axis=2), axis=1)[:, None]
            cy = jnp.sum(jnp.sum(jnp.where(m3, Y, 0.0), axis=2), axis=1)[:, None]
            cz = jnp.sum(jnp.sum(jnp.where(m3, Z, 0.0), axis=2), axis=1)[:, None]
            NX = jnp.where(lane1 == t, cx, NX)
            NY = jnp.where(lane1 == t, cy, NY)
            NZ = jnp.where(lane1 == t, cz, NZ)
            dx = X - cx[:, :, None]
            dy = Y - cy[:, :, None]
            dz = Z - cz[:, :, None]
            d = dx * dx + dy * dy + dz * dz
            dist = jnp.minimum(dist, d)
            mx = jnp.max(jnp.max(dist, axis=2), axis=1)[:, None]
            f_new = jnp.min(jnp.min(
                jnp.where(dist == mx[:, :, None], iota3, N), axis=2), axis=1)[:, None]
            return (dist, f_new, C, NX, NY, NZ)

        zi = jnp.where(row_n < 0, row_n, 0)
        zf = jnp.where(row_n < 0, 1.0, 0.0)
        init = (1e10 + 0.0 * X, jnp.zeros((B, 1), jnp.int32), zi, zf, zf, zf)
        _, _, C, NX, NY, NZ = lax.fori_loop(0, n_out, step, init)
        idx_ref[...] = C
        nx_ref[:, 0, :] = NX
        nx_ref[:, 1, :] = NY
        nx_ref[:, 2, :] = NZ

    return pl.pallas_call(
        body,
        out_shape=[jax.ShapeDtypeStruct((B, n_out), jnp.int32),
                   jax.ShapeDtypeStruct((B, 3, n_out), jnp.float32)],
    )(xr)


def _ball_query(new_xyz, xyz, xyzT, radius, nsample):
    B, S, _ = new_xyz.shape
    N = xyz.shape[1]
    r2 = radius ** 2

    def body(a_ref, b_ref, bt_ref, o_ref):
        a = a_ref[0]
        bm = b_ref[0]
        bt = bt_ref[0]
        sa = (a[:, 0:1] * a[:, 0:1] + a[:, 1:2] * a[:, 1:2]
              + a[:, 2:3] * a[:, 2:3])
        sb = (bt[0:1] * bt[0:1] + bt[1:2] * bt[1:2]
              + bt[2:3] * bt[2:3])
        d = lax.dot_general(a, bm, (((1,), (1,)), ((), ())),
                            preferred_element_type=jnp.float32)
        sq = sa + sb - 2.0 * d
        col = lax.broadcasted_iota(jnp.int32, (S, N), 1)
        keys = jnp.where(sq > r2, N, col)
        rowO = lax.broadcasted_iota(jnp.int32, (S, nsample), 0)
        colO = lax.broadcasted_iota(jnp.int32, (S, nsample), 1)
        O0 = jnp.where(rowO < 0, rowO, N)

        def cond(c):
            _, _, j, have = c
            return jnp.logical_and(j < nsample, have)

        def step(c):
            keys, O, j, _ = c
            m = jnp.min(keys, axis=1, keepdims=True)
            O = jnp.where(colO == j, m, O)
            keys = jnp.where(keys == m, N, keys)
            return (keys, O, j + 1, jnp.min(keys) < N)

        _, O, _, _ = lax.while_loop(
            cond, step, (keys, O0, jnp.int32(0), jnp.min(keys) < N))
        first = O[:, 0:1]
        o_ref[0] = jnp.where(O == N, first, O)

    return pl.pallas_call(
        body,
        grid=(B,),
        in_specs=[pl.BlockSpec((1, S, 3), lambda b: (b, 0, 0)),
                  pl.BlockSpec((1, N, 3), lambda b: (b, 0, 0)),
                  pl.BlockSpec((1, 3, N), lambda b: (b, 0, 0))],
        out_specs=pl.BlockSpec((1, S, nsample), lambda b: (b, 0, 0)),
        out_shape=jax.ShapeDtypeStruct((B, S, nsample), jnp.int32),
    )(new_xyz, xyz, xyzT)


def _sc_gather(table, idx):
    V, D = table.shape
    R = idx.shape[0]
    NC, NS = 2, 16
    CH = min(128, 65536 // (D * 4))
    NW = NC * NS
    npc = R // (NW * CH)
    assert R == NW * CH * npc, (R, D)
    mesh = plsc.VectorSubcoreMesh(core_axis_name="c", subcore_axis_name="s")

    @functools.partial(
        pl.kernel, mesh=mesh,
        out_type=jax.ShapeDtypeStruct((R, D), jnp.float32),
        scratch_types=[pltpu.VMEM((CH,), jnp.int32),
                       pltpu.VMEM((CH,), jnp.int32),
                       pltpu.VMEM((CH, D), jnp.float32),
                       pltpu.VMEM((CH, D), jnp.float32),
                       pltpu.SemaphoreType.DMA,
                       pltpu.SemaphoreType.DMA],
    )
    def k(tab, ix, out, ix_a, ix_b, rows_a, rows_b, sem_a, sem_b):
        wid = lax.axis_index("s") * NC + lax.axis_index("c")
        bufs = ((ix_a, rows_a, sem_a), (ix_b, rows_b, sem_b))

        pending = [None, None]
        for c in range(npc):
            ix_v, rows_v, sem = bufs[c % 2]
            base = pl.multiple_of((wid * npc + c) * CH, CH)
            pltpu.sync_copy(ix.at[pl.ds(base, CH)], ix_v)
            pending[c % 2] = pltpu.async_copy(tab.at[ix_v], rows_v, sem)
            if c >= 1:
                pix, prows, psem = bufs[(c - 1) % 2]
                pending[(c - 1) % 2].wait()
                pbase = pl.multiple_of((wid * npc + c - 1) * CH, CH)
                pltpu.sync_copy(prows, out.at[pl.ds(pbase, CH)])
        lix, lrows, lsem = bufs[(npc - 1) % 2]
        pending[(npc - 1) % 2].wait()
        lbase = pl.multiple_of((wid * npc + npc - 1) * CH, CH)
        pltpu.sync_copy(lrows, out.at[pl.ds(lbase, CH)])

    return k(table, idx)


def _bn(h, gamma, beta):
    return (h / _BN_C) * gamma + beta


def _mlp_pool1(grouped, new_xyz, w):
    B, RS, _ = grouped.shape
    S = new_xyz.shape[1]
    K = RS // S
    (w1, b1, g1, e1), (w2, b2, g2, e2), (w3, b3, g3, e3) = w
    co = w3.shape[1]

    def body(g_ref, nx_ref, w1r, b1r, g1r, e1r, w2r, b2r, g2r, e2r,
             w3r, b3r, g3r, e3r, o_ref):
        nx = nx_ref[0]
        W1, W2, W3 = w1r[...], w2r[...], w3r[...]
        B1, G1, E1 = b1r[...], g1r[...], e1r[...]
        B2, G2, E2 = b2r[...], g2r[...], e2r[...]
        B3, G3, E3 = b3r[...], g3r[...], e3r[...]

        def one(k):
            g = g_ref[0, pl.ds(k * S, S), :]
            xc = g[:, 0:3] - nx
            h = jax.nn.relu(_bn(lax.dot_general(
                xc, W1, (((1,), (0,)), ((), ())),
                preferred_element_type=jnp.float32) + B1, G1, E1))
            h = jax.nn.relu(_bn(lax.dot_general(
                h, W2, (((1,), (0,)), ((), ())),
                preferred_element_type=jnp.float32) + B2, G2, E2))
            h = jax.nn.relu(_bn(lax.dot_general(
                h, W3, (((1,), (0,)), ((), ())),
                preferred_element_type=jnp.float32) + B3, G3, E3))
            return h

        o_ref[0] = lax.fori_loop(
            1, K, lambda k, acc: jnp.maximum(acc, one(k)), one(0))

    wspecs = []
    wargs = []
    for (W, bb, gg, ee) in ((w1, b1, g1, e1), (w2, b2, g2, e2), (w3, b3, g3, e3)):
        for arr in (W, bb, gg, ee):
            wspecs.append(pl.BlockSpec(arr.shape, lambda b: (0,) * arr.ndim))
            wargs.append(arr)
    return pl.pallas_call(
        body,
        grid=(B,),
        in_specs=[pl.BlockSpec((1, RS, grouped.shape[2]), lambda b: (b, 0, 0)),
                  pl.BlockSpec((1, S, 3), lambda b: (b, 0, 0))] + wspecs,
        out_specs=pl.BlockSpec((1, S, co), lambda b: (b, 0, 0)),
        out_shape=jax.ShapeDtypeStruct((B, S, co), jnp.float32),
    )(grouped, new_xyz, *wargs)


def _mlp_pool2(grouped, new_xyz, w):
    B, RS, DD = grouped.shape
    S = new_xyz.shape[1]
    K = RS // S
    (w1x, w1f, b1, g1, e1), (w2, b2, g2, e2), (w3, b3, g3, e3) = w
    co = w3.shape[1]

    def body(g_ref, nx_ref, w1xr, w1fr, b1r, g1r, e1r, w2r, b2r, g2r, e2r,
             w3r, b3r, g3r, e3r, o_ref):
        nx = nx_ref[0]
        W1x, W1f = w1xr[...], w1fr[...]
        W2, W3 = w2r[...], w3r[...]
        B1, G1, E1 = b1r[...], g1r[...], e1r[...]
        B2, G2, E2 = b2r[...], g2r[...], e2r[...]
        B3, G3, E3 = b3r[...], g3r[...], e3r[...]

        def one(k):
            g = g_ref[0, pl.ds(k * S, S), :]
            F = g[:, 0:128]
            xc = g[:, 128:131] - nx
            h = lax.dot_general(xc, W1x, (((1,), (0,)), ((), ())),
                                preferred_element_type=jnp.float32)
            h = h + lax.dot_general(F, W1f, (((1,), (0,)), ((), ())),
                                    preferred_element_type=jnp.float32)
            h = jax.nn.relu(_bn(h + B1, G1, E1))
            h = jax.nn.relu(_bn(lax.dot_general(
                h, W2, (((1,), (0,)), ((), ())),
                preferred_element_type=jnp.float32) + B2, G2, E2))
            h = jax.nn.relu(_bn(lax.dot_general(
                h, W3, (((1,), (0,)), ((), ())),
                preferred_element_type=jnp.float32) + B3, G3, E3))
            return h

        o_ref[0] = lax.fori_loop(
            1, K, lambda k, acc: jnp.maximum(acc, one(k)), one(0))

    wspecs = []
    wargs = []
    for arr in (w1x, w1f, b1, g1, e1, w2, b2, g2, e2, w3, b3, g3, e3):
        wspecs.append(pl.BlockSpec(arr.shape, lambda b: (0,) * arr.ndim))
        wargs.append(arr)
    return pl.pallas_call(
        body,
        grid=(B,),
        in_specs=[pl.BlockSpec((1, RS, DD), lambda b: (b, 0, 0)),
                  pl.BlockSpec((1, S, 3), lambda b: (b, 0, 0))] + wspecs,
        out_specs=pl.BlockSpec((1, S, co), lambda b: (b, 0, 0)),
        out_shape=jax.ShapeDtypeStruct((B, S, co), jnp.float32),
    )(grouped, new_xyz, *wargs)


def _sa3(l2_xyz, l2_points, w):
    B, S, _ = l2_xyz.shape
    (w1x, w1f, b1, g1, e1), (w2, b2, g2, e2), (w3, b3, g3, e3) = w
    co = w3.shape[1]

    def body(x_ref, p_ref, w1xr, w1fr, b1r, g1r, e1r, w2r, b2r, g2r, e2r,
             w3r, b3r, g3r, e3r, o_ref):
        xyz = x_ref[0]
        pts = p_ref[0]
        h = lax.dot_general(xyz, w1xr[...], (((1,), (0,)), ((), ())),
                            preferred_element_type=jnp.float32)
        h = h + lax.dot_general(pts, w1fr[...], (((1,), (0,)), ((), ())),
                                preferred_element_type=jnp.float32)
        h = jax.nn.relu(_bn(h + b1r[...], g1r[...], e1r[...]))
        h = jax.nn.relu(_bn(lax.dot_general(
            h, w2r[...], (((1,), (0,)), ((), ())),
            preferred_element_type=jnp.float32) + b2r[...], g2r[...], e2r[...]))
        h = jax.nn.relu(_bn(lax.dot_general(
            h, w3r[...], (((1,), (0,)), ((), ())),
            preferred_element_type=jnp.float32) + b3r[...], g3r[...], e3r[...]))
        o_ref[0] = jnp.max(h, axis=0, keepdims=True)

    wspecs = []
    wargs = []
    for arr in (w1x, w1f, b1, g1, e1, w2, b2, g2, e2, w3, b3, g3, e3):
        wspecs.append(pl.BlockSpec(arr.shape, lambda b: (0,) * arr.ndim))
        wargs.append(arr)
    return pl.pallas_call(
        body,
        grid=(B,),
        in_specs=[pl.BlockSpec((1, S, 3), lambda b: (b, 0, 0)),
                  pl.BlockSpec((1, S, l2_points.shape[2]), lambda b: (b, 0, 0))]
                 + wspecs,
        out_specs=pl.BlockSpec((1, 1, co), lambda b: (b, 0, 0)),
        out_shape=jax.ShapeDtypeStruct((B, 1, co), jnp.float32),
    )(l2_xyz, l2_points, *wargs)


def _prep_layer1(lp):
    out = []
    for (W, b, gm, bt) in lp:
        co = W.shape[0]
        out.append((W.T, b.reshape(1, co), gm.reshape(1, co), bt.reshape(1, co)))
    return out


def _prep_layer_split(lp, nx):
    (W1, b1, gm1, bt1) = lp[0]
    co1 = W1.shape[0]
    if nx == 0:
        w1x = W1[:, 0:3].T
        w1f = W1[:, 3:].T
    else:
        w1x = W1[:, 0:3].T
        w1f = W1[:, 3:].T
    first = (w1x, w1f, b1.reshape(1, co1), gm1.reshape(1, co1), bt1.reshape(1, co1))
    rest = []
    for (W, b, gm, bt) in lp[1:]:
        co = W.shape[0]
        rest.append((W.T, b.reshape(1, co), gm.reshape(1, co), bt.reshape(1, co)))
    return (first, rest[0], rest[1])


def kernel(x, params):
    B, N, _ = x.shape
    p1, p2, p3 = params

    xT = x.transpose(0, 2, 1)
    xr1 = xT.reshape(B, 3, 8, N // 8)

    fps1_idx, nxyzT1 = _fps(xr1, 512)
    del fps1_idx
    new_xyz1 = nxyzT1.transpose(0, 2, 1)
    idx1 = _ball_query(new_xyz1, x, xT, 0.03, 32)

    table1 = jnp.pad(x.reshape(B * N, 3), ((0, 0), (0, 125)))
    gi1 = (jnp.minimum(idx1, N - 1).transpose(0, 2, 1)
           + (jnp.arange(B, dtype=jnp.int32) * N)[:, None, None])
    g1 = _sc_gather(table1, gi1.reshape(-1).astype(jnp.int32))
    grouped1 = g1.reshape(B, 32 * 512, 128)
    l1_points = _mlp_pool1(grouped1, new_xyz1, _prep_layer1(p1))

    xr2 = nxyzT1.reshape(B, 3, 8, 64)
    fps2_idx, nxyzT2 = _fps(xr2, 128)
    del fps2_idx
    new_xyz2 = nxyzT2.transpose(0, 2, 1)
    idx2 = _ball_query(new_xyz2, new_xyz1, nxyzT1, 0.06, 64)

    table2 = jnp.concatenate(
        [l1_points.reshape(B * 512, 128),
         new_xyz1.reshape(B * 512, 3),
         jnp.zeros((B * 512, 125), jnp.float32)], axis=1)
    gi2 = (jnp.minimum(idx2, 511).transpose(0, 2, 1)
           + (jnp.arange(B, dtype=jnp.int32) * 512)[:, None, None])
    g2 = _sc_gather(table2, gi2.reshape(-1).astype(jnp.int32))
    grouped2 = g2.reshape(B, 64 * 128, 256)
    l2_points = _mlp_pool2(grouped2, new_xyz2, _prep_layer_split(p2, 0))

    out = _sa3(new_xyz2, l2_points, _prep_layer_split(p3, 0))
    return out.reshape(B, 1024)


# speedup vs baseline: 6.4581x; 1.0011x over previous
import functools

import numpy as np
import jax
import jax.numpy as jnp
from jax import lax
from jax.experimental import pallas as pl
from jax.experimental.pallas import tpu as pltpu
from jax.experimental.pallas import tpu_sc as plsc

_BN_C = np.sqrt(np.float32(1.0 + 1e-5)).astype(np.float32)


def _fps(xr, n_out):
    B, _, S, L = xr.shape
    N = S * L

    def body(xr_ref, idx_ref, nx_ref):
        X = xr_ref[:, 0]
        Y = xr_ref[:, 1]
        Z = xr_ref[:, 2]
        iota3 = (lax.broadcasted_iota(jnp.int32, (B, S, L), 1) * L
                 + lax.broadcasted_iota(jnp.int32, (B, S, L), 2))
        row_n = lax.broadcasted_iota(jnp.int32, (B, n_out), 0)
        lane1 = lax.broadcasted_iota(jnp.int32, (B, n_out), 1)

        def step(t, carry):
            dist, f, C, NX, NY, NZ = carry
            C = jnp.where(lane1 == t, f, C)
            m3 = iota3 == f[:, :, None]
            cx = jnp.sum(jnp.sum(jnp.where(m3, X, 0.0), axis=2), axis=1)[:, None]
            cy = jnp.sum(jnp.sum(jnp.where(m3, Y, 0.0), axis=2), axis=1)[:, None]
            cz = jnp.sum(jnp.sum(jnp.where(m3, Z, 0.0), axis=2), axis=1)[:, None]
            NX = jnp.where(lane1 == t, cx, NX)
            NY = jnp.where(lane1 == t, cy, NY)
            NZ = jnp.where(lane1 == t, cz, NZ)
            dx = X - cx[:, :, None]
            dy = Y - cy[:, :, None]
            dz = Z - cz[:, :, None]
            d = dx * dx + dy * dy + dz * dz
            dist = jnp.minimum(dist, d)
            mx = jnp.max(jnp.max(dist, axis=2), axis=1)[:, None]
            f_new = jnp.min(jnp.min(
                jnp.where(dist == mx[:, :, None], iota3, N), axis=2), axis=1)[:, None]
            return (dist, f_new, C, NX, NY, NZ)

        zi = jnp.where(row_n < 0, row_n, 0)
        zf = jnp.where(row_n < 0, 1.0, 0.0)
        init = (1e10 + 0.0 * X, jnp.zeros((B, 1), jnp.int32), zi, zf, zf, zf)
        _, _, C, NX, NY, NZ = lax.fori_loop(0, n_out, step, init)
        idx_ref[...] = C
        nx_ref[:, 0, :] = NX
        nx_ref[:, 1, :] = NY
        nx_ref[:, 2, :] = NZ

    return pl.pallas_call(
        body,
        out_shape=[jax.ShapeDtypeStruct((B, n_out), jnp.int32),
                   jax.ShapeDtypeStruct((B, 3, n_out), jnp.float32)],
    )(xr)


def _ball_query(new_xyz, xyz, xyzT, radius, nsample):
    B, S, _ = new_xyz.shape
    N = xyz.shape[1]
    r2 = radius ** 2

    def body(a_ref, b_ref, bt_ref, o_ref):
        a = a_ref[0]
        bm = b_ref[0]
        bt = bt_ref[0]
        sa = (a[:, 0:1] * a[:, 0:1] + a[:, 1:2] * a[:, 1:2]
              + a[:, 2:3] * a[:, 2:3])
        sb = (bt[0:1] * bt[0:1] + bt[1:2] * bt[1:2]
              + bt[2:3] * bt[2:3])
        d = lax.dot_general(a, bm, (((1,), (1,)), ((), ())),
                            preferred_element_type=jnp.float32)
        sq = sa + sb - 2.0 * d
        col = lax.broadcasted_iota(jnp.int32, (S, N), 1)
        keys = jnp.where(sq > r2, N, col)
        rowO = lax.broadcasted_iota(jnp.int32, (S, nsample), 0)
        colO = lax.broadcasted_iota(jnp.int32, (S, nsample), 1)
        O0 = jnp.where(rowO < 0, rowO, N)

        def cond(c):
            _, _, j, have = c
            return jnp.logical_and(j < nsample, have)

        def step(c):
            keys, O, j, _ = c
            m = jnp.min(keys, axis=1, keepdims=True)
            O = jnp.where(colO == j, m, O)
            keys = jnp.where(keys == m, N, keys)
            return (keys, O, j + 1, jnp.min(keys) < N)

        _, O, _, _ = lax.while_loop(
            cond, step, (keys, O0, jnp.int32(0), jnp.min(keys) < N))
        first = O[:, 0:1]
        o_ref[0] = jnp.where(O == N, first, O)

    return pl.pallas_call(
        body,
        grid=(B,),
        in_specs=[pl.BlockSpec((1, S, 3), lambda b: (b, 0, 0)),
                  pl.BlockSpec((1, N, 3), lambda b: (b, 0, 0)),
                  pl.BlockSpec((1, 3, N), lambda b: (b, 0, 0))],
        out_specs=pl.BlockSpec((1, S, nsample), lambda b: (b, 0, 0)),
        out_shape=jax.ShapeDtypeStruct((B, S, nsample), jnp.int32),
    )(new_xyz, xyz, xyzT)


def _sc_gather(table, idx):
    V, D = table.shape
    R = idx.shape[0]
    NC, NS = 2, 16
    CH = min(128, 65536 // (D * 4))
    NW = NC * NS
    npc = R // (NW * CH)
    assert R == NW * CH * npc, (R, D)
    mesh = plsc.VectorSubcoreMesh(core_axis_name="c", subcore_axis_name="s")

    @functools.partial(
        pl.kernel, mesh=mesh,
        out_type=jax.ShapeDtypeStruct((R, D), jnp.float32),
        scratch_types=[pltpu.VMEM((CH,), jnp.int32),
                       pltpu.VMEM((CH,), jnp.int32),
                       pltpu.VMEM((CH, D), jnp.float32),
                       pltpu.VMEM((CH, D), jnp.float32),
                       pltpu.SemaphoreType.DMA,
                       pltpu.SemaphoreType.DMA],
    )
    def k(tab, ix, out, ix_a, ix_b, rows_a, rows_b, sem_a, sem_b):
        wid = lax.axis_index("s") * NC + lax.axis_index("c")
        bufs = ((ix_a, rows_a, sem_a), (ix_b, rows_b, sem_b))

        pending = [None, None]
        for c in range(npc):
            ix_v, rows_v, sem = bufs[c % 2]
            base = pl.multiple_of((wid * npc + c) * CH, CH)
            pltpu.sync_copy(ix.at[pl.ds(base, CH)], ix_v)
            pending[c % 2] = pltpu.async_copy(tab.at[ix_v], rows_v, sem)
            if c >= 1:
                pix, prows, psem = bufs[(c - 1) % 2]
                pending[(c - 1) % 2].wait()
                pbase = pl.multiple_of((wid * npc + c - 1) * CH, CH)
                pltpu.sync_copy(prows, out.at[pl.ds(pbase, CH)])
        lix, lrows, lsem = bufs[(npc - 1) % 2]
        pending[(npc - 1) % 2].wait()
        lbase = pl.multiple_of((wid * npc + npc - 1) * CH, CH)
        pltpu.sync_copy(lrows, out.at[pl.ds(lbase, CH)])

    return k(table, idx)


def _bn(h, gamma, beta):
    return (h / _BN_C) * gamma + beta


def _mlp_pool1(grouped, new_xyz, w):
    B, RS, _ = grouped.shape
    S = new_xyz.shape[1]
    K = RS // S
    (w1, b1, g1, e1), (w2, b2, g2, e2), (w3, b3, g3, e3) = w
    co = w3.shape[1]

    def body(g_ref, nx_ref, w1r, b1r, g1r, e1r, w2r, b2r, g2r, e2r,
             w3r, b3r, g3r, e3r, o_ref):
        nx = nx_ref[0]
        W1, W2, W3 = w1r[...], w2r[...], w3r[...]
        B1, G1, E1 = b1r[...], g1r[...], e1r[...]
        B2, G2, E2 = b2r[...], g2r[...], e2r[...]
        B3, G3, E3 = b3r[...], g3r[...], e3r[...]

        def one(k):
            g = g_ref[0, pl.ds(k * S, S), :]
            xc = g[:, 0:3] - nx
            h = jax.nn.relu(_bn(lax.dot_general(
                xc, W1, (((1,), (0,)), ((), ())),
                preferred_element_type=jnp.float32) + B1, G1, E1))
            h = jax.nn.relu(_bn(lax.dot_general(
                h, W2, (((1,), (0,)), ((), ())),
                preferred_element_type=jnp.float32) + B2, G2, E2))
            h = jax.nn.relu(_bn(lax.dot_general(
                h, W3, (((1,), (0,)), ((), ())),
                preferred_element_type=jnp.float32) + B3, G3, E3))
            return h

        o_ref[0] = lax.fori_loop(
            1, K, lambda k, acc: jnp.maximum(acc, one(k)), one(0))

    wspecs = []
    wargs = []
    for (W, bb, gg, ee) in ((w1, b1, g1, e1), (w2, b2, g2, e2), (w3, b3, g3, e3)):
        for arr in (W, bb, gg, ee):
            wspecs.append(pl.BlockSpec(arr.shape, lambda b: (0,) * arr.ndim))
            wargs.append(arr)
    return pl.pallas_call(
        body,
        grid=(B,),
        in_specs=[pl.BlockSpec((1, RS, grouped.shape[2]), lambda b: (b, 0, 0)),
                  pl.BlockSpec((1, S, 3), lambda b: (b, 0, 0))] + wspecs,
        out_specs=pl.BlockSpec((1, S, co), lambda b: (b, 0, 0)),
        out_shape=jax.ShapeDtypeStruct((B, S, co), jnp.float32),
    )(grouped, new_xyz, *wargs)


def _mlp_pool2(grouped, new_xyz, w):
    B, RS, DD = grouped.shape
    S = new_xyz.shape[1]
    K = RS // S
    (w1x, w1f, b1, g1, e1), (w2, b2, g2, e2), (w3, b3, g3, e3) = w
    co = w3.shape[1]

    def body(g_ref, nx_ref, w1xr, w1fr, b1r, g1r, e1r, w2r, b2r, g2r, e2r,
             w3r, b3r, g3r, e3r, o_ref):
        nx = nx_ref[0]
        W1x, W1f = w1xr[...], w1fr[...]
        W2, W3 = w2r[...], w3r[...]
        B1, G1, E1 = b1r[...], g1r[...], e1r[...]
        B2, G2, E2 = b2r[...], g2r[...], e2r[...]
        B3, G3, E3 = b3r[...], g3r[...], e3r[...]

        def one(k):
            g = g_ref[0, pl.ds(k * S, S), :]
            F = g[:, 0:128]
            xc = g[:, 128:131] - nx
            h = lax.dot_general(xc, W1x, (((1,), (0,)), ((), ())),
                                preferred_element_type=jnp.float32)
            h = h + lax.dot_general(F, W1f, (((1,), (0,)), ((), ())),
                                    preferred_element_type=jnp.float32)
            h = jax.nn.relu(_bn(h + B1, G1, E1))
            h = jax.nn.relu(_bn(lax.dot_general(
                h, W2, (((1,), (0,)), ((), ())),
                preferred_element_type=jnp.float32) + B2, G2, E2))
            h = jax.nn.relu(_bn(lax.dot_general(
                h, W3, (((1,), (0,)), ((), ())),
                preferred_element_type=jnp.float32) + B3, G3, E3))
            return h

        o_ref[0] = lax.fori_loop(
            1, K, lambda k, acc: jnp.maximum(acc, one(k)), one(0))

    wspecs = []
    wargs = []
    for arr in (w1x, w1f, b1, g1, e1, w2, b2, g2, e2, w3, b3, g3, e3):
        wspecs.append(pl.BlockSpec(arr.shape, lambda b: (0,) * arr.ndim))
        wargs.append(arr)
    return pl.pallas_call(
        body,
        grid=(B,),
        in_specs=[pl.BlockSpec((1, RS, DD), lambda b: (b, 0, 0)),
                  pl.BlockSpec((1, S, 3), lambda b: (b, 0, 0))] + wspecs,
        out_specs=pl.BlockSpec((1, S, co), lambda b: (b, 0, 0)),
        out_shape=jax.ShapeDtypeStruct((B, S, co), jnp.float32),
    )(grouped, new_xyz, *wargs)


def _sa3(l2_xyz, l2_points, w):
    B, S, _ = l2_xyz.shape
    (w1x, w1f, b1, g1, e1), (w2, b2, g2, e2), (w3, b3, g3, e3) = w
    co = w3.shape[1]

    def body(x_ref, p_ref, w1xr, w1fr, b1r, g1r, e1r, w2r, b2r, g2r, e2r,
             w3r, b3r, g3r, e3r, o_ref):
        xyz = x_ref[0]
        pts = p_ref[0]
        h = lax.dot_general(xyz, w1xr[...], (((1,), (0,)), ((), ())),
                            preferred_element_type=jnp.float32)
        h = h + lax.dot_general(pts, w1fr[...], (((1,), (0,)), ((), ())),
                                preferred_element_type=jnp.float32)
        h = jax.nn.relu(_bn(h + b1r[...], g1r[...], e1r[...]))
        h = jax.nn.relu(_bn(lax.dot_general(
            h, w2r[...], (((1,), (0,)), ((), ())),
            preferred_element_type=jnp.float32) + b2r[...], g2r[...], e2r[...]))
        h = jax.nn.relu(_bn(lax.dot_general(
            h, w3r[...], (((1,), (0,)), ((), ())),
            preferred_element_type=jnp.float32) + b3r[...], g3r[...], e3r[...]))
        o_ref[0] = jnp.max(h, axis=0, keepdims=True)

    wspecs = []
    wargs = []
    for arr in (w1x, w1f, b1, g1, e1, w2, b2, g2, e2, w3, b3, g3, e3):
        wspecs.append(pl.BlockSpec(arr.shape, lambda b: (0,) * arr.ndim))
        wargs.append(arr)
    return pl.pallas_call(
        body,
        grid=(B,),
        in_specs=[pl.BlockSpec((1, S, 3), lambda b: (b, 0, 0)),
                  pl.BlockSpec((1, S, l2_points.shape[2]), lambda b: (b, 0, 0))]
                 + wspecs,
        out_specs=pl.BlockSpec((1, 1, co), lambda b: (b, 0, 0)),
        out_shape=jax.ShapeDtypeStruct((B, 1, co), jnp.float32),
    )(l2_xyz, l2_points, *wargs)


def _prep_layer1(lp):
    out = []
    for (W, b, gm, bt) in lp:
        co = W.shape[0]
        out.append((W.T, b.reshape(1, co), gm.reshape(1, co), bt.reshape(1, co)))
    return out


def _prep_layer_split(lp, nx):
    (W1, b1, gm1, bt1) = lp[0]
    co1 = W1.shape[0]
    if nx == 0:
        w1x = W1[:, 0:3].T
        w1f = W1[:, 3:].T
    else:
        w1x = W1[:, 0:3].T
        w1f = W1[:, 3:].T
    first = (w1x, w1f, b1.reshape(1, co1), gm1.reshape(1, co1), bt1.reshape(1, co1))
    rest = []
    for (W, b, gm, bt) in lp[1:]:
        co = W.shape[0]
        rest.append((W.T, b.reshape(1, co), gm.reshape(1, co), bt.reshape(1, co)))
    return (first, rest[0], rest[1])


def kernel(x, params):
    B, N, _ = x.shape
    p1, p2, p3 = params

    xT = x.transpose(0, 2, 1)
    xr1 = xT.reshape(B, 3, 8, N // 8)

    fps1_idx, nxyzT1 = _fps(xr1, 512)
    del fps1_idx
    new_xyz1 = nxyzT1.transpose(0, 2, 1)
    idx1 = _ball_query(new_xyz1, x, xT, 0.03, 32)

    table1 = jnp.pad(x.reshape(B * N, 3), ((0, 0), (0, 125)))
    gi1 = (jnp.minimum(idx1, N - 1).transpose(0, 2, 1)
           + (jnp.arange(B, dtype=jnp.int32) * N)[:, None, None])
    g1 = _sc_gather(table1, gi1.reshape(-1).astype(jnp.int32))

    xr2 = nxyzT1.reshape(B, 3, 8, 64)
    fps2_idx, nxyzT2 = _fps(xr2, 128)
    del fps2_idx
    new_xyz2 = nxyzT2.transpose(0, 2, 1)
    idx2 = _ball_query(new_xyz2, new_xyz1, nxyzT1, 0.06, 64)

    grouped1 = g1.reshape(B, 32 * 512, 128)
    l1_points = _mlp_pool1(grouped1, new_xyz1, _prep_layer1(p1))

    table2 = jnp.concatenate(
        [l1_points.reshape(B * 512, 128),
         new_xyz1.reshape(B * 512, 3),
         jnp.zeros((B * 512, 125), jnp.float32)], axis=1)
    gi2 = (jnp.minimum(idx2, 511).transpose(0, 2, 1)
           + (jnp.arange(B, dtype=jnp.int32) * 512)[:, None, None])
    g2 = _sc_gather(table2, gi2.reshape(-1).astype(jnp.int32))
    grouped2 = g2.reshape(B, 64 * 128, 256)
    l2_points = _mlp_pool2(grouped2, new_xyz2, _prep_layer_split(p2, 0))

    out = _sa3(new_xyz2, l2_points, _prep_layer_split(p3, 0))
    return out.reshape(B, 1024)
